# Initial kernel scaffold; baseline (speedup 1.0000x reference)
import jax
import jax.numpy as jnp
from jax.experimental import pallas as pl


def kernel(x, edge_index, W1, b1, W2, b2):
    raise NotImplementedError("write your pallas kernel here")


# speedup vs baseline: 11.6865x; 11.6865x over previous
import functools

import jax
import jax.numpy as jnp
from jax import lax
from jax.experimental import pallas as pl
from jax.experimental.pallas import tpu as pltpu
from jax.experimental.pallas import tpu_sc as plsc

NC = 2
NS = 16
CH = 128
BR = 400


def _cdiv(a, b):
  return (a + b - 1) // b


def _make_deg_kernel(N, E, NP):
  EPW = E // (NC * NS)
  NCHW = _cdiv(EPW, CH)
  WR = NP // NS
  mesh = plsc.VectorSubcoreMesh(core_axis_name="c", subcore_axis_name="s")

  @functools.partial(
      pl.kernel,
      mesh=mesh,
      out_type=jax.ShapeDtypeStruct((NC, 1, NP), jnp.float32),
      scratch_types=[
          pltpu.VMEM((NCHW, CH), jnp.int32),
          pltpu.VMEM((CH,), jnp.float32),
          pltpu.VMEM_SHARED((NP,), jnp.float32),
      ],
  )
  def deg_kernel(dstc_hbm, zeros_hbm, out_hbm, idx_v, ones_v, acc):
    c = lax.axis_index("c")
    s = lax.axis_index("s")
    pltpu.sync_copy(dstc_hbm.at[c, s], idx_v)
    for i in range(CH // 16):
      ones_v[pl.ds(i * 16, 16)] = jnp.ones((16,), jnp.float32)

    @pl.when(s == 0)
    def _():
      pltpu.sync_copy(zeros_hbm, acc)

    plsc.subcore_barrier()

    def body(j, carry):
      pltpu.sync_copy(ones_v, acc.at[idx_v.at[j]], add=True)
      return carry

    lax.fori_loop(0, NCHW, body, 0)
    plsc.subcore_barrier()
    pltpu.sync_copy(acc.at[pl.ds(s * WR, WR)],
                    out_hbm.at[c, 0, pl.ds(s * WR, WR)])

  return deg_kernel


def _make_prop_kernel(N, NP, Dc, K, split_cols):
  RPT = NP // NS
  mesh = plsc.VectorSubcoreMesh(core_axis_name="c", subcore_axis_name="s")

  @functools.partial(
      pl.kernel,
      mesh=mesh,
      out_type=jax.ShapeDtypeStruct((NC, NP, Dc), jnp.float32),
      scratch_types=[
          pltpu.VMEM((K, CH), jnp.int32),
          pltpu.VMEM((K, CH), jnp.int32),
          pltpu.VMEM((CH, Dc), jnp.float32),
          pltpu.VMEM_SHARED((NP, Dc), jnp.float32),
          pltpu.SemaphoreType.DMA,
      ],
  )
  def prop_kernel(table_hbm, srcc_hbm, dstc_hbm, out_hbm,
                  src_v, dst_v, rows_v, acc, sem):
    c = lax.axis_index("c")
    s = lax.axis_index("s")
    pltpu.sync_copy(srcc_hbm.at[c, s], src_v)
    pltpu.sync_copy(dstc_hbm.at[c, s], dst_v)
    base = c * NP + s * RPT if split_cols else s * RPT
    pltpu.sync_copy(table_hbm.at[pl.ds(base, RPT)], acc.at[pl.ds(s * RPT, RPT)])
    plsc.subcore_barrier()

    def body(j, carry):
      pltpu.async_copy(table_hbm.at[src_v.at[j]], rows_v, sem).wait()
      pltpu.sync_copy(rows_v, acc.at[dst_v.at[j]], add=True)
      return carry

    lax.fori_loop(0, K, body, 0)
    plsc.subcore_barrier()
    pltpu.sync_copy(acc.at[pl.ds(s * RPT, RPT)],
                    out_hbm.at[c, pl.ds(s * RPT, RPT)])

  return prop_kernel


def _scale_call(degT, x, N, NP, DIN):
  Dc = DIN // 2

  def body(degT_ref, x_ref, u1_ref, dinv_ref):
    deg = degT_ref[:, 0:1] + degT_ref[:, 1:2] + 1.0
    dv = lax.rsqrt(deg)
    u = x_ref[...] * dv
    u1_ref[0] = u[:, :Dc]
    u1_ref[1] = u[:, Dc:]
    dinv_ref[...] = dv

  return pl.pallas_call(
      body,
      grid=(_cdiv(NP, BR),),
      in_specs=[
          pl.BlockSpec((BR, 2), lambda i: (i, 0)),
          pl.BlockSpec((BR, DIN), lambda i: (i, 0)),
      ],
      out_specs=[
          pl.BlockSpec((NC, BR, Dc), lambda i: (0, i, 0)),
          pl.BlockSpec((BR, 1), lambda i: (i, 0)),
      ],
      out_shape=[
          jax.ShapeDtypeStruct((NC, NP, Dc), jnp.float32),
          jax.ShapeDtypeStruct((N, 1), jnp.float32),
      ],
  )(degT, x)


def _mlp_call(acc1, dinv, W1, b1, W2, N, NP, DIN, DHID, DOUT):
  Dc = DIN // 2

  def body(acc_ref, dinv_ref, w1_ref, b1_ref, w2_ref, u2_ref):
    dv = dinv_ref[...]
    a0 = acc_ref[0] * dv
    a1 = acc_ref[1] * dv
    w1 = w1_ref[...]
    t = (jnp.dot(a0, w1[:Dc, :], preferred_element_type=jnp.float32)
         + jnp.dot(a1, w1[Dc:, :], preferred_element_type=jnp.float32)
         + b1_ref[...])
    h = jnp.maximum(t, 0.0)
    u2_ref[...] = jnp.dot(h, w2_ref[...],
                          preferred_element_type=jnp.float32) * dv

  return pl.pallas_call(
      body,
      grid=(_cdiv(NP, BR),),
      in_specs=[
          pl.BlockSpec((NC, BR, Dc), lambda i: (0, i, 0)),
          pl.BlockSpec((BR, 1), lambda i: (i, 0)),
          pl.BlockSpec((DIN, DHID), lambda i: (0, 0)),
          pl.BlockSpec((1, DHID), lambda i: (0, 0)),
          pl.BlockSpec((DHID, DOUT), lambda i: (0, 0)),
      ],
      out_specs=pl.BlockSpec((BR, DOUT), lambda i: (i, 0)),
      out_shape=jax.ShapeDtypeStruct((NP, DOUT), jnp.float32),
  )(acc1, dinv, W1, b1, W2)


def _final_call(acc2, u2, dinv, b2, N, NP, DOUT):
  def body(acc_ref, u2_ref, dinv_ref, b2_ref, out_ref):
    dv = dinv_ref[...]
    o = acc_ref[0] + acc_ref[1] - u2_ref[...]
    out_ref[...] = o * dv + b2_ref[...]

  return pl.pallas_call(
      body,
      grid=(N // BR,),
      in_specs=[
          pl.BlockSpec((NC, BR, DOUT), lambda i: (0, i, 0)),
          pl.BlockSpec((BR, DOUT), lambda i: (i, 0)),
          pl.BlockSpec((BR, 1), lambda i: (i, 0)),
          pl.BlockSpec((1, DOUT), lambda i: (0, 0)),
      ],
      out_specs=pl.BlockSpec((BR, DOUT), lambda i: (i, 0)),
      out_shape=jax.ShapeDtypeStruct((N, DOUT), jnp.float32),
  )(acc2, u2, dinv, b2)


def kernel(x, edge_index, W1, b1, W2, b2):
  N, DIN = x.shape
  E = edge_index.shape[1]
  DHID = W1.shape[1]
  DOUT = W2.shape[1]
  src = edge_index[0]
  dst = edge_index[1]

  NP = _cdiv(N + 1, 128 * NS) * 128 * NS

  EPW = E // (NC * NS)
  NCHW = _cdiv(EPW, CH)
  padw = NCHW * CH - EPW
  dst_w = jnp.pad(dst.reshape(NC, NS, EPW), ((0, 0), (0, 0), (0, padw)),
                  constant_values=N).reshape(NC, NS, NCHW, CH)
  src_w = jnp.pad(src.reshape(NC, NS, EPW), ((0, 0), (0, 0), (0, padw)),
                  constant_values=0).reshape(NC, NS, NCHW, CH)
  EPT = E // NS
  NCH = _cdiv(EPT, CH)
  padt = NCH * CH - EPT
  srcr = jnp.pad(src.reshape(NS, EPT), ((0, 0), (0, padt)),
                 constant_values=0).reshape(NS, NCH, CH)
  src_t = jnp.stack([srcr, srcr + NP])
  dstr = jnp.pad(dst.reshape(NS, EPT), ((0, 0), (0, padt)),
                 constant_values=N).reshape(NS, NCH, CH)
  dst_t = jnp.stack([dstr, dstr])

  zeros = jnp.zeros((NP,), jnp.float32)

  deg_p = _make_deg_kernel(N, E, NP)(dst_w, zeros)
  degT = deg_p[:, 0, :N].T
  u1, dinv = _scale_call(degT, x, N, NP, DIN)
  acc1 = _make_prop_kernel(N, NP, DIN // 2, NCH, True)(
      u1.reshape(2 * NP, DIN // 2), src_t, dst_t)
  u2 = _mlp_call(acc1, dinv, W1, b1.reshape(1, DHID), W2, N, NP, DIN, DHID,
                 DOUT)
  acc2 = _make_prop_kernel(N, NP, DOUT, NCHW, False)(u2, src_w, dst_w)
  return _final_call(acc2, u2, dinv, b2.reshape(1, DOUT), N, NP, DOUT)
